# Initial kernel scaffold; baseline (speedup 1.0000x reference)
import jax
import jax.numpy as jnp
from jax.experimental import pallas as pl


def kernel(x, Wg, bg, We, be):
    raise NotImplementedError("write your pallas kernel here")


# speedup vs baseline: 6.0770x; 6.0770x over previous
import functools

import jax
import jax.numpy as jnp
from jax.experimental import pallas as pl
from jax.experimental.pallas import tpu as pltpu

N, S, D = 1, 2048, 768
OUT = 768
E = 8
TOP_K = 2

TILE_S = 256


def _moe_dense_kernel(x_ref, wg_ref, bg_ref, we_ref, be_ref, out_ref):
    x = x_ref[...]
    logits = jax.lax.dot_general(
        x, wg_ref[...], (((1,), (1,)), ((), ())),
        preferred_element_type=jnp.float32)
    logits = logits + bg_ref[...]

    lane = jax.lax.broadcasted_iota(jnp.int32, (TILE_S, E), 1)
    big = jnp.int32(E)
    m1 = jnp.max(logits, axis=1, keepdims=True)
    a1 = jnp.min(jnp.where(logits == m1, lane, big), axis=1, keepdims=True)
    neg = jnp.float32(-jnp.inf)
    logits2 = jnp.where(lane == a1, neg, logits)
    m2 = jnp.max(logits2, axis=1, keepdims=True)
    a2 = jnp.min(jnp.where(logits2 == m2, lane, big), axis=1, keepdims=True)
    mask = ((lane == a1) | (lane == a2)).astype(jnp.float32)

    acc = jnp.zeros((TILE_S, OUT), dtype=jnp.float32)
    for e in range(E):
        y = jax.lax.dot_general(
            x, we_ref[e], (((1,), (0,)), ((), ())),
            preferred_element_type=jnp.float32)
        y = jnp.maximum(y + be_ref[e][None, :], 0.0)
        acc = acc + mask[:, e][:, None] * y
    out_ref[...] = acc * jnp.float32(1.0 / TOP_K)


def kernel(x, Wg, bg, We, be):
    x2 = x.reshape(S, D)
    bg2 = bg.reshape(1, E)
    grid = (S // TILE_S,)
    out = pl.pallas_call(
        _moe_dense_kernel,
        grid=grid,
        in_specs=[
            pl.BlockSpec((TILE_S, D), lambda i: (i, 0)),
            pl.BlockSpec((E, D), lambda i: (0, 0)),
            pl.BlockSpec((1, E), lambda i: (0, 0)),
            pl.BlockSpec((E, D, OUT), lambda i: (0, 0, 0)),
            pl.BlockSpec((E, OUT), lambda i: (0, 0)),
        ],
        out_specs=pl.BlockSpec((TILE_S, OUT), lambda i: (i, 0)),
        out_shape=jax.ShapeDtypeStruct((S, OUT), jnp.float32),
    )(x2, Wg, bg2, We, be)
    return out.reshape(N, S, OUT)
